# baseline (speedup 1.0000x reference)
import functools

import jax
import jax.numpy as jnp
from jax import lax
from jax.experimental import pallas as pl
from jax.experimental.pallas import tpu as pltpu
from jax.experimental.pallas import tpu_sc as plsc

_NC = 2
_NS = 16
_NW = _NC * _NS

_SEQ = 8192
_EMBED = 768
_ROWS_PER_W = _SEQ // _NW
_CHUNK = 64
_NBUF = 2
_NCHUNKS = _ROWS_PER_W // _CHUNK


def _sc_copy_body(table_hbm, out_hbm, buf, in_sems, out_sems):
    wid = lax.axis_index("s") * _NC + lax.axis_index("c")
    base = wid * _ROWS_PER_W
    in_copies = []
    out_copies = []
    for i in range(_NCHUNKS):
        src = table_hbm.at[pl.ds(base + i * _CHUNK, _CHUNK)]
        dst = out_hbm.at[pl.ds(base + i * _CHUNK, _CHUNK)]
        in_copies.append(
            pltpu.make_async_copy(src, buf.at[i % _NBUF], in_sems.at[i % _NBUF]))
        out_copies.append(
            pltpu.make_async_copy(buf.at[i % _NBUF], dst, out_sems.at[i % _NBUF]))
    for i in range(_NBUF):
        in_copies[i].start()
    for i in range(_NCHUNKS):
        in_copies[i].wait()
        out_copies[i].start()
        if i + _NBUF < _NCHUNKS:
            out_copies[i].wait()
            in_copies[i + _NBUF].start()
    for i in range(max(0, _NCHUNKS - _NBUF), _NCHUNKS):
        out_copies[i].wait()


@functools.partial(
    pl.kernel,
    mesh=plsc.VectorSubcoreMesh(core_axis_name="c", subcore_axis_name="s"),
    out_type=jax.ShapeDtypeStruct((_SEQ, _EMBED), jnp.float32),
    scratch_types=[
        pltpu.VMEM((_NBUF, _CHUNK, _EMBED), jnp.float32),
        pltpu.SemaphoreType.DMA((_NBUF,)),
        pltpu.SemaphoreType.DMA((_NBUF,)),
    ],
)
def _sc_copy(table_hbm, out_hbm, buf, in_sems, out_sems):
    _sc_copy_body(table_hbm, out_hbm, buf, in_sems, out_sems)


def kernel(x, pos_table):
    del x
    return _sc_copy(pos_table)


# speedup vs baseline: 1.0222x; 1.0222x over previous
import functools

import jax
import jax.numpy as jnp
from jax import lax
from jax.experimental import pallas as pl
from jax.experimental.pallas import tpu as pltpu
from jax.experimental.pallas import tpu_sc as plsc

_NC = 2
_NS = 16
_NW = _NC * _NS

_SEQ = 8192
_EMBED = 768
_ROWS_PER_W = _SEQ // _NW
_CHUNK = 32
_NBUF = 4
_NCHUNKS = _ROWS_PER_W // _CHUNK


def _sc_copy_body(table_hbm, out_hbm, buf, in_sems, out_sems):
    wid = lax.axis_index("s") * _NC + lax.axis_index("c")
    base = wid * _ROWS_PER_W
    in_copies = []
    out_copies = []
    for i in range(_NCHUNKS):
        src = table_hbm.at[pl.ds(base + i * _CHUNK, _CHUNK)]
        dst = out_hbm.at[pl.ds(base + i * _CHUNK, _CHUNK)]
        in_copies.append(
            pltpu.make_async_copy(src, buf.at[i % _NBUF], in_sems.at[i % _NBUF]))
        out_copies.append(
            pltpu.make_async_copy(buf.at[i % _NBUF], dst, out_sems.at[i % _NBUF]))
    for i in range(_NBUF):
        in_copies[i].start()
    for i in range(_NCHUNKS):
        in_copies[i].wait()
        out_copies[i].start()
        if i + _NBUF < _NCHUNKS:
            out_copies[i].wait()
            in_copies[i + _NBUF].start()
    for i in range(max(0, _NCHUNKS - _NBUF), _NCHUNKS):
        out_copies[i].wait()


@functools.partial(
    pl.kernel,
    mesh=plsc.VectorSubcoreMesh(core_axis_name="c", subcore_axis_name="s"),
    out_type=jax.ShapeDtypeStruct((_SEQ, _EMBED), jnp.float32),
    scratch_types=[
        pltpu.VMEM((_NBUF, _CHUNK, _EMBED), jnp.float32),
        pltpu.SemaphoreType.DMA((_NBUF,)),
        pltpu.SemaphoreType.DMA((_NBUF,)),
    ],
)
def _sc_copy(table_hbm, out_hbm, buf, in_sems, out_sems):
    _sc_copy_body(table_hbm, out_hbm, buf, in_sems, out_sems)


def kernel(x, pos_table):
    del x
    return _sc_copy(pos_table)


# speedup vs baseline: 2.2981x; 2.2482x over previous
import jax
import jax.numpy as jnp
from jax.experimental import pallas as pl
from jax.experimental.pallas import tpu as pltpu

_BLK_ROWS = 4096


def _copy_body(table_ref, out_ref):
    out_ref[...] = table_ref[...]


def kernel(x, pos_table):
    seqlen = x.shape[-1]
    embed = pos_table.shape[1]
    nblk = pl.cdiv(seqlen, _BLK_ROWS)
    return pl.pallas_call(
        _copy_body,
        grid=(nblk,),
        in_specs=[pl.BlockSpec((_BLK_ROWS, embed), lambda i: (i, 0))],
        out_specs=pl.BlockSpec((_BLK_ROWS, embed), lambda i: (i, 0)),
        out_shape=jax.ShapeDtypeStruct((seqlen, embed), pos_table.dtype),
        compiler_params=pltpu.CompilerParams(
            dimension_semantics=("parallel",),
        ),
    )(pos_table)


# speedup vs baseline: 2.3011x; 1.0013x over previous
import jax
import jax.numpy as jnp
from jax.experimental import pallas as pl
from jax.experimental.pallas import tpu as pltpu

_BLK_ROWS = 4096


def _copy_body(table_ref, out_ref):
    out_ref[...] = table_ref[...]


def kernel(x, pos_table):
    seqlen = x.shape[-1]
    embed = pos_table.shape[1]
    nblk = pl.cdiv(seqlen, _BLK_ROWS)
    return pl.pallas_call(
        _copy_body,
        grid=(nblk,),
        in_specs=[pl.BlockSpec((_BLK_ROWS, embed), lambda i: (i, 0))],
        out_specs=pl.BlockSpec((_BLK_ROWS, embed), lambda i: (i, 0)),
        out_shape=jax.ShapeDtypeStruct((seqlen, embed), pos_table.dtype),
        compiler_params=pltpu.CompilerParams(
            dimension_semantics=("arbitrary",),
        ),
    )(pos_table)
